# Initial kernel scaffold; baseline (speedup 1.0000x reference)
import jax
import jax.numpy as jnp
from jax.experimental import pallas as pl


def kernel(x, target):
    raise NotImplementedError("write your pallas kernel here")


# speedup vs baseline: 1.8103x; 1.8103x over previous
import functools
import math

import jax
import jax.numpy as jnp
from jax import lax
from jax.experimental import pallas as pl
from jax.experimental.pallas import tpu as pltpu
from jax.experimental.pallas import tpu_sc as plsc

_N = 1024
_SIZE = 100000
_PAD = 0
_SMOOTH = 0.1
_CONF = 1.0 - _SMOOTH
_EPS = _SMOOTH / (_SIZE - 2)
_CCONST = (_SIZE - 2) * _EPS * math.log(_EPS) + _CONF * math.log(_CONF)

_BC = 2048
_NBLK = (_SIZE + _BC - 1) // _BC

_NTILES = 32
_RPT = _N // _NTILES


def _tc_body(x_ref, t_ref, out_ref, acc_ref):
    j = pl.program_id(0)
    nb = pl.num_programs(0)

    @pl.when(j == 0)
    def _():
        xb = x_ref[...]
        acc_ref[...] = x_ref[:, 0:1] - jnp.sum(xb, axis=1, keepdims=True)

    @pl.when(jnp.logical_and(j > 0, j < nb - 1))
    def _():
        acc_ref[...] -= jnp.sum(x_ref[...], axis=1, keepdims=True)

    @pl.when(j == nb - 1)
    def _():
        col = j * _BC + lax.broadcasted_iota(jnp.int32, (_N, _BC), 1)
        xm = jnp.where(col < _SIZE, x_ref[...], 0.0)
        acc = acc_ref[...] - jnp.sum(xm, axis=1, keepdims=True)
        valid = t_ref[...] != _PAD
        per_row = _CCONST + _EPS * acc
        total = jnp.sum(jnp.where(valid, per_row, 0.0))
        out_ref[...] = jnp.broadcast_to(total, (1, 1))


_tc_call = pl.pallas_call(
    _tc_body,
    grid=(_NBLK,),
    in_specs=[
        pl.BlockSpec((_N, _BC), lambda j: (0, j)),
        pl.BlockSpec((_N, 1), lambda j: (0, 0)),
    ],
    out_specs=pl.BlockSpec((1, 1), lambda j: (0, 0)),
    out_shape=jax.ShapeDtypeStruct((1, 1), jnp.float32),
    scratch_shapes=[pltpu.VMEM((_N, 1), jnp.float32)],
    compiler_params=pltpu.CompilerParams(
        dimension_semantics=("arbitrary",),
    ),
)


def _sc_gather_body(x_hbm, t_hbm, out_hbm, tv, rowbuf, accbuf):
    c = lax.axis_index("c")
    s = lax.axis_index("s")
    wid = s * 2 + c
    base = wid * _RPT
    pltpu.sync_copy(t_hbm.at[pl.ds(base, _RPT)], tv)
    iota = lax.broadcasted_iota(jnp.int32, (16,), 0)
    acc = jnp.zeros((16,), jnp.float32)
    for k in range(_RPT):
        t = tv[pl.ds((k // 16) * 16, 16)][k % 16]
        off = (t // 16) * 16
        pltpu.sync_copy(x_hbm.at[base + k, pl.ds(off, 16)], rowbuf)
        valid = jnp.minimum(jnp.abs(t), 1)
        ind = jnp.maximum(1 - jnp.abs(iota - (t - off)), 0) * valid
        acc = acc + rowbuf[...] * ind.astype(jnp.float32)
    accbuf[...] = acc * (_EPS - _CONF)
    pltpu.sync_copy(accbuf, out_hbm.at[pl.ds(wid * 16, 16)])


@functools.cache
def _get_sc_call():
    return functools.partial(
        pl.kernel,
        out_type=jax.ShapeDtypeStruct((_NTILES * 16,), jnp.float32),
        mesh=plsc.VectorSubcoreMesh(core_axis_name="c", subcore_axis_name="s"),
        scratch_types=[
            pltpu.VMEM((_RPT,), jnp.int32),
            pltpu.VMEM((16,), jnp.float32),
            pltpu.VMEM((16,), jnp.float32),
        ],
    )(_sc_gather_body)


def kernel(x, target):
    target = target.astype(jnp.int32)
    tc_out = _tc_call(x, target.reshape(_N, 1))
    sc_out = _get_sc_call()(x, target)
    return tc_out[0, 0] + jnp.sum(sc_out)
